# Initial kernel scaffold; baseline (speedup 1.0000x reference)
import functools

import jax
import jax.numpy as jnp
from jax import lax
from jax.experimental import pallas as pl
from jax.experimental.pallas import tpu as pltpu

BF = jnp.bfloat16
F32 = jnp.float32


def _round_up(x, m):
    return ((x + m - 1) // m) * m


def _silu(x):
    return x * jax.nn.sigmoid(x)


def _cparams(sems, vmem=None):
    kw = {"dimension_semantics": sems}
    if vmem is not None:
        kw["vmem_limit_bytes"] = vmem
    return pltpu.CompilerParams(**kw)


def _init_kernel(aid_ref, iid_ref, embA_ref, embI_ref, wtop_ref, wbot_ref,
                 b_ref, f32_ref, bf16_ref, *, n_atom_pad, n_id_pad):
    aid = aid_ref[...]
    iid = iid_ref[...]
    t = aid.shape[0]
    oh_a = (lax.broadcasted_iota(jnp.int32, (t, n_atom_pad), 1) == aid)
    oh_i = (lax.broadcasted_iota(jnp.int32, (t, n_id_pad), 1) == iid)
    p_top = jnp.dot(embI_ref[...], wtop_ref[...], preferred_element_type=F32)
    p_bot = jnp.dot(embA_ref[...], wbot_ref[...], preferred_element_type=F32)
    pre = jnp.dot(oh_i.astype(BF), p_top.astype(BF), preferred_element_type=F32)
    pre = pre + jnp.dot(oh_a.astype(BF), p_bot.astype(BF),
                        preferred_element_type=F32)
    out = _silu(pre + b_ref[...])
    f32_ref[...] = out
    bf16_ref[...] = out.astype(BF)


def _initial_feats(atomids, identity, embedding, embedding_id, w, b):
    n = atomids.shape[0]
    d = w.shape[1]
    eid = embedding_id.shape[1]
    na, ni = embedding.shape[0], embedding_id.shape[0]
    na_pad, ni_pad = _round_up(na, 8), _round_up(ni, 8)
    tn = 512 if n % 512 == 0 else n
    embA = jnp.pad(embedding, ((0, na_pad - na), (0, 0))).astype(BF)
    embI = jnp.pad(embedding_id, ((0, ni_pad - ni), (0, 0))).astype(BF)
    wtop = w[:eid].astype(BF)
    wbot = w[eid:].astype(BF)
    bb = b.reshape(1, -1).astype(F32)

    return pl.pallas_call(
        functools.partial(_init_kernel, n_atom_pad=na_pad, n_id_pad=ni_pad),
        out_shape=(jax.ShapeDtypeStruct((n, d), F32),
                   jax.ShapeDtypeStruct((n, d), BF)),
        grid=(n // tn,),
        in_specs=[pl.BlockSpec((tn, 1), lambda i: (i, 0)),
                  pl.BlockSpec((tn, 1), lambda i: (i, 0)),
                  pl.BlockSpec(embA.shape, lambda i: (0, 0)),
                  pl.BlockSpec(embI.shape, lambda i: (0, 0)),
                  pl.BlockSpec(wtop.shape, lambda i: (0, 0)),
                  pl.BlockSpec(wbot.shape, lambda i: (0, 0)),
                  pl.BlockSpec(bb.shape, lambda i: (0, 0))],
        out_specs=(pl.BlockSpec((tn, d), lambda i: (i, 0)),
                   pl.BlockSpec((tn, d), lambda i: (i, 0))),
        compiler_params=_cparams(("core_parallel",)),
        name="init_feats",
    )(atomids.reshape(n, 1).astype(jnp.int32),
      identity.reshape(n, 1).astype(jnp.int32), embA, embI, wtop, wbot, bb)


def _edge_kernel(seg_ref, xi_ref, xj_ref, rc_ref,
                 w1i_ref, w1j_ref, wf_ref, w2_ref, b2_ref,
                 wc1_ref, bc1_ref, wc2_ref, bc2_ref,
                 agg_ref, *, fourier_features, n_nodes, m_dim):
    step = pl.program_id(1)

    @pl.when(step == 0)
    def _init():
        agg_ref[...] = jnp.zeros_like(agg_ref)

    xi = xi_ref[...]
    xj = xj_ref[...]
    rel = rc_ref[...]
    te = rel.shape[0]
    d2 = jnp.sum(rel * rel, axis=-1, keepdims=True)

    scales = (0.5 ** jnp.arange(fourier_features, dtype=F32)).reshape(1, -1)
    dk = d2 * scales
    ff = jnp.concatenate(
        [jnp.sin(dk), jnp.cos(dk), d2, jnp.ones_like(d2),
         jnp.zeros((te, 16 - 2 * fourier_features - 2), F32)],
        axis=1).astype(BF)

    pre = jnp.dot(xi, w1i_ref[...], preferred_element_type=F32)
    pre = pre + jnp.dot(xj, w1j_ref[...], preferred_element_type=F32)
    pre = pre + jnp.dot(ff, wf_ref[...], preferred_element_type=F32)
    h = _silu(pre).astype(BF)

    m_ij = _silu(jnp.dot(h, w2_ref[...], preferred_element_type=F32)
                 + b2_ref[...])
    mb = m_ij.astype(BF)
    ch = _silu(jnp.dot(mb, wc1_ref[...], preferred_element_type=F32)
               + bc1_ref[...])
    cw = jnp.sum(ch * wc2_ref[...], axis=-1, keepdims=True) + bc2_ref[...]

    vals = jnp.concatenate(
        [mb, (cw * rel).astype(BF), jnp.ones((te, 1), BF),
         jnp.zeros((te, 12), BF)], axis=1)

    seg = seg_ref[...]
    one_hot = (lax.broadcasted_iota(jnp.int32, (n_nodes, te), 0)
               == seg).astype(BF)
    agg_ref[0] += jnp.dot(one_hot, vals, preferred_element_type=F32)


def _edge_aggregate(fb16, coors, src, dst, wp, *, fourier_features, m_dim):
    n, d = fb16.shape
    e = src.shape[0]
    te = 512
    n_cores = 2 if (e // te) % 2 == 0 else 1
    s = e // te // n_cores

    xi = fb16[dst]
    xj = fb16[src]
    rc = coors[src] - coors[dst]
    seg = dst.reshape(1, e).astype(jnp.int32)

    agg = pl.pallas_call(
        functools.partial(_edge_kernel, fourier_features=fourier_features,
                          n_nodes=n, m_dim=m_dim),
        out_shape=jax.ShapeDtypeStruct((n_cores, n, m_dim + 16), F32),
        grid=(n_cores, s),
        in_specs=[pl.BlockSpec((1, te), lambda c, i: (0, c * s + i)),
                  pl.BlockSpec((te, d), lambda c, i: (c * s + i, 0)),
                  pl.BlockSpec((te, d), lambda c, i: (c * s + i, 0)),
                  pl.BlockSpec((te, 3), lambda c, i: (c * s + i, 0))] +
                 [pl.BlockSpec(w.shape, lambda c, i: (0, 0))
                  for w in wp],
        out_specs=pl.BlockSpec((1, n, m_dim + 16), lambda c, i: (c, 0, 0)),
        compiler_params=_cparams(("core_parallel", "arbitrary"),
                                 vmem=48 * 1024 * 1024),
        name="edge_agg",
    )(seg, xi, xj, rc, *wp)
    return agg


def _node_kernel(feats_ref, coors_ref, agg_ref,
                 wn1f_ref, wn1m_ref, bn1_ref, wn2_ref, bn2_ref,
                 f32_ref, bf16_ref, co_ref, *, m_dim, n_cores):
    agg = agg_ref[0]
    for c in range(1, n_cores):
        agg = agg + agg_ref[c]
    summ = agg[:, :m_dim]
    rest = agg[:, m_dim:]
    cnt = rest[:, 3:4]
    inv = 1.0 / jnp.maximum(cnt, 1.0)
    agg_m = summ * inv
    agg_c = rest[:, :3] * inv

    feats = feats_ref[...]
    pre = jnp.dot(feats.astype(BF), wn1f_ref[...], preferred_element_type=F32)
    pre = pre + jnp.dot(agg_m.astype(BF), wn1m_ref[...],
                        preferred_element_type=F32)
    h = _silu(pre + bn1_ref[...]).astype(BF)
    out = feats + jnp.dot(h, wn2_ref[...], preferred_element_type=F32) \
        + bn2_ref[...]
    f32_ref[...] = out
    bf16_ref[...] = out.astype(BF)
    co_ref[...] = coors_ref[...] + agg_c


def _node_update(feats, coors, agg, wp, *, m_dim):
    n, d = feats.shape
    n_cores = agg.shape[0]
    aw = agg.shape[2]
    tn = 512 if n % 512 == 0 else n

    return pl.pallas_call(
        functools.partial(_node_kernel, m_dim=m_dim, n_cores=n_cores),
        out_shape=(jax.ShapeDtypeStruct((n, d), F32),
                   jax.ShapeDtypeStruct((n, d), BF),
                   jax.ShapeDtypeStruct((n, 3), F32)),
        grid=(n // tn,),
        in_specs=[pl.BlockSpec((tn, d), lambda i: (i, 0)),
                  pl.BlockSpec((tn, 3), lambda i: (i, 0)),
                  pl.BlockSpec((n_cores, tn, aw), lambda i: (0, i, 0))] +
                 [pl.BlockSpec(w.shape, lambda i: (0, 0)) for w in wp],
        out_specs=(pl.BlockSpec((tn, d), lambda i: (i, 0)),
                   pl.BlockSpec((tn, d), lambda i: (i, 0)),
                   pl.BlockSpec((tn, 3), lambda i: (i, 0))),
        compiler_params=_cparams(("core_parallel",)),
        name="node_update",
    )(feats, coors, agg, *wp)


def _fnn_kernel(f0_ref, f1_ref, f2_ref, f3_ref,
                w0_ref, b0_ref, w1_ref, b1_ref, w2_ref, b2_ref, o_ref):
    x = jnp.concatenate([_silu(f0_ref[...]), _silu(f1_ref[...]),
                         _silu(f2_ref[...]), _silu(f3_ref[...])],
                        axis=1).astype(BF)
    h = _silu(jnp.dot(x, w0_ref[...], preferred_element_type=F32)
              + b0_ref[...]).astype(BF)
    h = _silu(jnp.dot(h, w1_ref[...], preferred_element_type=F32)
              + b1_ref[...]).astype(BF)
    h = _silu(jnp.dot(h, w2_ref[...], preferred_element_type=F32)
              + b2_ref[...])
    o_ref[...] = h.astype(BF)


def _fnn_stack(flist, wp):
    n, d = flist[0].shape
    out_dim = wp[4].shape[1]
    tn = 512 if n % 512 == 0 else n
    return pl.pallas_call(
        _fnn_kernel,
        out_shape=jax.ShapeDtypeStruct((n, out_dim), BF),
        grid=(n // tn,),
        in_specs=[pl.BlockSpec((tn, d), lambda i: (i, 0))
                  for _ in flist] +
                 [pl.BlockSpec(w.shape, lambda i: (0, 0)) for w in wp],
        out_specs=pl.BlockSpec((tn, out_dim), lambda i: (i, 0)),
        compiler_params=_cparams(("core_parallel",)),
        name="fnn_stack",
    )(*flist, *wp)


def _head_kernel(seg_ref, h_ref, w0_ref, b0_ref, w1_ref, b1_ref,
                 w2_ref, b2_ref, o_ref, acc_ref, cnt_ref, *, num_graphs):
    step = pl.program_id(0)

    @pl.when(step == 0)
    def _init():
        acc_ref[...] = jnp.zeros_like(acc_ref)
        cnt_ref[...] = jnp.zeros_like(cnt_ref)

    seg = seg_ref[...]
    one_hot = (lax.broadcasted_iota(jnp.int32, (num_graphs, seg.shape[1]), 0)
               == seg)
    acc_ref[...] += jnp.dot(one_hot.astype(BF), h_ref[...],
                            preferred_element_type=F32)
    cnt_ref[...] += jnp.sum(one_hot.astype(F32), axis=-1, keepdims=True)

    @pl.when(step == pl.num_programs(0) - 1)
    def _fin():
        g = acc_ref[...] * (1.0 / jnp.maximum(cnt_ref[...], 1.0))
        g = _silu(jnp.dot(g.astype(BF), w0_ref[...],
                          preferred_element_type=F32) + b0_ref[...])
        g = _silu(jnp.dot(g.astype(BF), w1_ref[...],
                          preferred_element_type=F32) + b1_ref[...])
        o_ref[...] = jnp.sum(g * w2_ref[...], axis=-1, keepdims=True) \
            + b2_ref[...]


def _graph_head(h, batch, wp, *, num_graphs):
    n, feat = h.shape
    tn = 1024 if n % 1024 == 0 else n
    seg = batch.reshape(1, n).astype(jnp.int32)
    return pl.pallas_call(
        functools.partial(_head_kernel, num_graphs=num_graphs),
        out_shape=jax.ShapeDtypeStruct((num_graphs, 1), F32),
        grid=(n // tn,),
        in_specs=[pl.BlockSpec((1, tn), lambda i: (0, i)),
                  pl.BlockSpec((tn, feat), lambda i: (i, 0))] +
                 [pl.BlockSpec(w.shape, lambda i: (0, 0)) for w in wp],
        out_specs=pl.BlockSpec((num_graphs, 1), lambda i: (0, 0)),
        scratch_shapes=[pltpu.VMEM((num_graphs, feat), F32),
                        pltpu.VMEM((num_graphs, 1), F32)],
        compiler_params=_cparams(("arbitrary",)),
        name="graph_head",
    )(seg, h, *wp)


def _prep_edge_weights(e1w, e1b, e2w, e2b, c1w, c1b, c2w, c2b, *, d, ff):
    h1_raw = e1w.shape[1]
    h1 = _round_up(h1_raw, 128)
    e1wp = jnp.pad(e1w, ((0, 0), (0, h1 - h1_raw)))
    b1p = jnp.pad(e1b, (0, h1 - h1_raw)).reshape(1, h1)
    w1i = e1wp[:d].astype(BF)
    w1j = e1wp[d:2 * d].astype(BF)
    frows = e1wp[2 * d:2 * d + 2 * ff + 1]
    wf = jnp.concatenate(
        [frows, b1p, jnp.zeros((16 - (2 * ff + 2), h1), F32)],
        axis=0).astype(BF)
    w2p = jnp.pad(e2w, ((0, h1 - h1_raw), (0, 0))).astype(BF)
    return [w1i, w1j, wf, w2p, e2b.reshape(1, -1).astype(F32),
            c1w.astype(BF), c1b.reshape(1, -1).astype(F32),
            c2w.reshape(1, -1).astype(F32), c2b.reshape(1, 1).astype(F32)]


def _prep_node_weights(n1w, n1b, n2w, n2b, *, d):
    return [n1w[:d].astype(BF), n1w[d:].astype(BF),
            n1b.reshape(1, -1).astype(F32), n2w.astype(BF),
            n2b.reshape(1, -1).astype(F32)]


def kernel(atomids, identity, coords, edge_index, batch,
           embedding, embedding_id, initialfnn_w, initialfnn_b,
           k0_edge1_w, k0_edge1_b, k0_edge2_w, k0_edge2_b,
           k0_coors1_w, k0_coors1_b, k0_coors2_w, k0_coors2_b,
           k0_node1_w, k0_node1_b, k0_node2_w, k0_node2_b,
           k1_edge1_w, k1_edge1_b, k1_edge2_w, k1_edge2_b,
           k1_coors1_w, k1_coors1_b, k1_coors2_w, k1_coors2_b,
           k1_node1_w, k1_node1_b, k1_node2_w, k1_node2_b,
           k2_edge1_w, k2_edge1_b, k2_edge2_w, k2_edge2_b,
           k2_coors1_w, k2_coors1_b, k2_coors2_w, k2_coors2_b,
           k2_node1_w, k2_node1_b, k2_node2_w, k2_node2_b,
           f0_w, f0_b, f1_w, f1_b, f2_w, f2_b,
           g0_w, g0_b, g1_w, g1_b, g2_w, g2_b):
    d = initialfnn_w.shape[1]
    m_dim = k0_coors1_w.shape[0]
    ff = (k0_edge1_w.shape[0] - 2 * d - 1) // 2
    num_graphs = 64

    src = edge_index[0]
    dst = edge_index[1]
    coors = coords.astype(F32)

    feats, fb16 = _initial_feats(atomids, identity, embedding, embedding_id,
                                 initialfnn_w, initialfnn_b)

    layers = [
        (_prep_edge_weights(k0_edge1_w, k0_edge1_b, k0_edge2_w, k0_edge2_b,
                            k0_coors1_w, k0_coors1_b, k0_coors2_w, k0_coors2_b,
                            d=d, ff=ff),
         _prep_node_weights(k0_node1_w, k0_node1_b, k0_node2_w, k0_node2_b,
                            d=d)),
        (_prep_edge_weights(k1_edge1_w, k1_edge1_b, k1_edge2_w, k1_edge2_b,
                            k1_coors1_w, k1_coors1_b, k1_coors2_w, k1_coors2_b,
                            d=d, ff=ff),
         _prep_node_weights(k1_node1_w, k1_node1_b, k1_node2_w, k1_node2_b,
                            d=d)),
        (_prep_edge_weights(k2_edge1_w, k2_edge1_b, k2_edge2_w, k2_edge2_b,
                            k2_coors1_w, k2_coors1_b, k2_coors2_w, k2_coors2_b,
                            d=d, ff=ff),
         _prep_node_weights(k2_node1_w, k2_node1_b, k2_node2_w, k2_node2_b,
                            d=d)),
    ]

    flist = [feats]
    for ew, nw in layers:
        agg = _edge_aggregate(fb16, coors, src, dst, ew,
                              fourier_features=ff, m_dim=m_dim)
        feats, fb16, coors = _node_update(feats, coors, agg, nw, m_dim=m_dim)
        flist.append(feats)

    fnn_w = [f0_w.astype(BF), f0_b.reshape(1, -1).astype(F32),
             f1_w.astype(BF), f1_b.reshape(1, -1).astype(F32),
             f2_w.astype(BF), f2_b.reshape(1, -1).astype(F32)]
    h = _fnn_stack(flist, fnn_w)

    head_w = [g0_w.astype(BF), g0_b.reshape(1, -1).astype(F32),
              g1_w.astype(BF), g1_b.reshape(1, -1).astype(F32),
              g2_w.reshape(1, -1).astype(F32), g2_b.reshape(1, 1).astype(F32)]
    return _graph_head(h, batch, head_w, num_graphs=num_graphs)


# speedup vs baseline: 1.4528x; 1.4528x over previous
import functools

import jax
import jax.numpy as jnp
from jax import lax
from jax.experimental import pallas as pl
from jax.experimental.pallas import tpu as pltpu

BF = jnp.bfloat16
F32 = jnp.float32


def _round_up(x, m):
    return ((x + m - 1) // m) * m


def _silu(x):
    return x * jax.nn.sigmoid(x)


def _cparams(sems, vmem=None):
    kw = {"dimension_semantics": sems}
    if vmem is not None:
        kw["vmem_limit_bytes"] = vmem
    return pltpu.CompilerParams(**kw)


def _init_kernel(aid_ref, iid_ref, embA_ref, embI_ref, wtop_ref, wbot_ref,
                 b_ref, f32_ref, bf16_ref, *, n_atom_pad, n_id_pad):
    aid = aid_ref[...]
    iid = iid_ref[...]
    t = aid.shape[0]
    oh_a = (lax.broadcasted_iota(jnp.int32, (t, n_atom_pad), 1) == aid)
    oh_i = (lax.broadcasted_iota(jnp.int32, (t, n_id_pad), 1) == iid)
    p_top = jnp.dot(embI_ref[...], wtop_ref[...], preferred_element_type=F32)
    p_bot = jnp.dot(embA_ref[...], wbot_ref[...], preferred_element_type=F32)
    pre = jnp.dot(oh_i.astype(BF), p_top.astype(BF), preferred_element_type=F32)
    pre = pre + jnp.dot(oh_a.astype(BF), p_bot.astype(BF),
                        preferred_element_type=F32)
    out = _silu(pre + b_ref[...])
    f32_ref[...] = out
    bf16_ref[...] = out.astype(BF)


def _initial_feats(atomids, identity, embedding, embedding_id, w, b):
    n = atomids.shape[0]
    d = w.shape[1]
    eid = embedding_id.shape[1]
    na, ni = embedding.shape[0], embedding_id.shape[0]
    na_pad, ni_pad = _round_up(na, 8), _round_up(ni, 8)
    tn = 512 if n % 512 == 0 else n
    embA = jnp.pad(embedding, ((0, na_pad - na), (0, 0))).astype(BF)
    embI = jnp.pad(embedding_id, ((0, ni_pad - ni), (0, 0))).astype(BF)
    wtop = w[:eid].astype(BF)
    wbot = w[eid:].astype(BF)
    bb = b.reshape(1, -1).astype(F32)

    return pl.pallas_call(
        functools.partial(_init_kernel, n_atom_pad=na_pad, n_id_pad=ni_pad),
        out_shape=(jax.ShapeDtypeStruct((n, d), F32),
                   jax.ShapeDtypeStruct((n, d), BF)),
        grid=(n // tn,),
        in_specs=[pl.BlockSpec((tn, 1), lambda i: (i, 0)),
                  pl.BlockSpec((tn, 1), lambda i: (i, 0)),
                  pl.BlockSpec(embA.shape, lambda i: (0, 0)),
                  pl.BlockSpec(embI.shape, lambda i: (0, 0)),
                  pl.BlockSpec(wtop.shape, lambda i: (0, 0)),
                  pl.BlockSpec(wbot.shape, lambda i: (0, 0)),
                  pl.BlockSpec(bb.shape, lambda i: (0, 0))],
        out_specs=(pl.BlockSpec((tn, d), lambda i: (i, 0)),
                   pl.BlockSpec((tn, d), lambda i: (i, 0))),
        compiler_params=_cparams(("arbitrary",)),
        name="init_feats",
    )(atomids.reshape(n, 1).astype(jnp.int32),
      identity.reshape(n, 1).astype(jnp.int32), embA, embI, wtop, wbot, bb)


def _edge_kernel(seg_ref, xi_ref, xj_ref, rc_ref,
                 w1i_ref, w1j_ref, wf_ref, w2_ref, b2_ref,
                 wc1_ref, bc1_ref, wc2_ref, bc2_ref,
                 agg_ref, *, fourier_features, n_nodes, m_dim):
    step = pl.program_id(0)

    @pl.when(step == 0)
    def _init():
        agg_ref[...] = jnp.zeros_like(agg_ref)

    xi = xi_ref[...]
    xj = xj_ref[...]
    rel = rc_ref[...]
    te = rel.shape[0]
    d2 = jnp.sum(rel * rel, axis=-1, keepdims=True)

    dks = [d2 * (0.5 ** k) for k in range(fourier_features)]
    ff = jnp.concatenate(
        [jnp.sin(jnp.concatenate(dks, axis=1)),
         jnp.cos(jnp.concatenate(dks, axis=1)), d2, jnp.ones_like(d2),
         jnp.zeros((te, 16 - 2 * fourier_features - 2), F32)],
        axis=1).astype(BF)

    pre = jnp.dot(xi, w1i_ref[...], preferred_element_type=F32)
    pre = pre + jnp.dot(xj, w1j_ref[...], preferred_element_type=F32)
    pre = pre + jnp.dot(ff, wf_ref[...], preferred_element_type=F32)
    h = _silu(pre).astype(BF)

    m_ij = _silu(jnp.dot(h, w2_ref[...], preferred_element_type=F32)
                 + b2_ref[...])
    mb = m_ij.astype(BF)
    ch = _silu(jnp.dot(mb, wc1_ref[...], preferred_element_type=F32)
               + bc1_ref[...])
    cw = jnp.sum(ch * wc2_ref[...], axis=-1, keepdims=True) + bc2_ref[...]

    vals = jnp.concatenate(
        [mb, (cw * rel).astype(BF), jnp.ones((te, 1), BF),
         jnp.zeros((te, 12), BF)], axis=1)

    seg = seg_ref[...]
    one_hot = (lax.broadcasted_iota(jnp.int32, (n_nodes, te), 0)
               == seg).astype(BF)
    agg_ref[0] += jnp.dot(one_hot, vals, preferred_element_type=F32)


def _edge_aggregate(fb16, coors, src, dst, wp, *, fourier_features, m_dim):
    n, d = fb16.shape
    e = src.shape[0]
    te = 512
    n_cores = 1
    s = e // te

    xi = fb16[dst]
    xj = fb16[src]
    rc = coors[src] - coors[dst]
    seg = dst.reshape(1, e).astype(jnp.int32)

    agg = pl.pallas_call(
        functools.partial(_edge_kernel, fourier_features=fourier_features,
                          n_nodes=n, m_dim=m_dim),
        out_shape=jax.ShapeDtypeStruct((n_cores, n, m_dim + 16), F32),
        grid=(s,),
        in_specs=[pl.BlockSpec((1, te), lambda i: (0, i)),
                  pl.BlockSpec((te, d), lambda i: (i, 0)),
                  pl.BlockSpec((te, d), lambda i: (i, 0)),
                  pl.BlockSpec((te, 3), lambda i: (i, 0))] +
                 [pl.BlockSpec(w.shape, lambda i: (0, 0))
                  for w in wp],
        out_specs=pl.BlockSpec((1, n, m_dim + 16), lambda i: (0, 0, 0)),
        compiler_params=_cparams(("arbitrary",),
                                 vmem=48 * 1024 * 1024),
        name="edge_agg",
    )(seg, xi, xj, rc, *wp)
    return agg


def _node_kernel(feats_ref, coors_ref, agg_ref,
                 wn1f_ref, wn1m_ref, bn1_ref, wn2_ref, bn2_ref,
                 f32_ref, bf16_ref, co_ref, *, m_dim, n_cores):
    agg = agg_ref[0]
    for c in range(1, n_cores):
        agg = agg + agg_ref[c]
    summ = agg[:, :m_dim]
    rest = agg[:, m_dim:]
    cnt = rest[:, 3:4]
    inv = 1.0 / jnp.maximum(cnt, 1.0)
    agg_m = summ * inv
    agg_c = rest[:, :3] * inv

    feats = feats_ref[...]
    pre = jnp.dot(feats.astype(BF), wn1f_ref[...], preferred_element_type=F32)
    pre = pre + jnp.dot(agg_m.astype(BF), wn1m_ref[...],
                        preferred_element_type=F32)
    h = _silu(pre + bn1_ref[...]).astype(BF)
    out = feats + jnp.dot(h, wn2_ref[...], preferred_element_type=F32) \
        + bn2_ref[...]
    f32_ref[...] = out
    bf16_ref[...] = out.astype(BF)
    co_ref[...] = coors_ref[...] + agg_c


def _node_update(feats, coors, agg, wp, *, m_dim):
    n, d = feats.shape
    n_cores = agg.shape[0]
    aw = agg.shape[2]
    tn = 512 if n % 512 == 0 else n

    return pl.pallas_call(
        functools.partial(_node_kernel, m_dim=m_dim, n_cores=n_cores),
        out_shape=(jax.ShapeDtypeStruct((n, d), F32),
                   jax.ShapeDtypeStruct((n, d), BF),
                   jax.ShapeDtypeStruct((n, 3), F32)),
        grid=(n // tn,),
        in_specs=[pl.BlockSpec((tn, d), lambda i: (i, 0)),
                  pl.BlockSpec((tn, 3), lambda i: (i, 0)),
                  pl.BlockSpec((n_cores, tn, aw), lambda i: (0, i, 0))] +
                 [pl.BlockSpec(w.shape, lambda i: (0, 0)) for w in wp],
        out_specs=(pl.BlockSpec((tn, d), lambda i: (i, 0)),
                   pl.BlockSpec((tn, d), lambda i: (i, 0)),
                   pl.BlockSpec((tn, 3), lambda i: (i, 0))),
        compiler_params=_cparams(("arbitrary",)),
        name="node_update",
    )(feats, coors, agg, *wp)


def _fnn_kernel(f0_ref, f1_ref, f2_ref, f3_ref,
                w0_ref, b0_ref, w1_ref, b1_ref, w2_ref, b2_ref, o_ref):
    x = jnp.concatenate([_silu(f0_ref[...]), _silu(f1_ref[...]),
                         _silu(f2_ref[...]), _silu(f3_ref[...])],
                        axis=1).astype(BF)
    h = _silu(jnp.dot(x, w0_ref[...], preferred_element_type=F32)
              + b0_ref[...]).astype(BF)
    h = _silu(jnp.dot(h, w1_ref[...], preferred_element_type=F32)
              + b1_ref[...]).astype(BF)
    h = _silu(jnp.dot(h, w2_ref[...], preferred_element_type=F32)
              + b2_ref[...])
    o_ref[...] = h.astype(BF)


def _fnn_stack(flist, wp):
    n, d = flist[0].shape
    out_dim = wp[4].shape[1]
    tn = 512 if n % 512 == 0 else n
    return pl.pallas_call(
        _fnn_kernel,
        out_shape=jax.ShapeDtypeStruct((n, out_dim), BF),
        grid=(n // tn,),
        in_specs=[pl.BlockSpec((tn, d), lambda i: (i, 0))
                  for _ in flist] +
                 [pl.BlockSpec(w.shape, lambda i: (0, 0)) for w in wp],
        out_specs=pl.BlockSpec((tn, out_dim), lambda i: (i, 0)),
        compiler_params=_cparams(("arbitrary",)),
        name="fnn_stack",
    )(*flist, *wp)


def _head_kernel(seg_ref, h_ref, w0_ref, b0_ref, w1_ref, b1_ref,
                 w2_ref, b2_ref, o_ref, acc_ref, cnt_ref, *, num_graphs):
    step = pl.program_id(0)

    @pl.when(step == 0)
    def _init():
        acc_ref[...] = jnp.zeros_like(acc_ref)
        cnt_ref[...] = jnp.zeros_like(cnt_ref)

    seg = seg_ref[...]
    one_hot = (lax.broadcasted_iota(jnp.int32, (num_graphs, seg.shape[1]), 0)
               == seg)
    acc_ref[...] += jnp.dot(one_hot.astype(BF), h_ref[...],
                            preferred_element_type=F32)
    cnt_ref[...] += jnp.sum(one_hot.astype(F32), axis=-1, keepdims=True)

    @pl.when(step == pl.num_programs(0) - 1)
    def _fin():
        g = acc_ref[...] * (1.0 / jnp.maximum(cnt_ref[...], 1.0))
        g = _silu(jnp.dot(g.astype(BF), w0_ref[...],
                          preferred_element_type=F32) + b0_ref[...])
        g = _silu(jnp.dot(g.astype(BF), w1_ref[...],
                          preferred_element_type=F32) + b1_ref[...])
        o_ref[...] = jnp.sum(g * w2_ref[...], axis=-1, keepdims=True) \
            + b2_ref[...]


def _graph_head(h, batch, wp, *, num_graphs):
    n, feat = h.shape
    tn = 1024 if n % 1024 == 0 else n
    seg = batch.reshape(1, n).astype(jnp.int32)
    return pl.pallas_call(
        functools.partial(_head_kernel, num_graphs=num_graphs),
        out_shape=jax.ShapeDtypeStruct((num_graphs, 1), F32),
        grid=(n // tn,),
        in_specs=[pl.BlockSpec((1, tn), lambda i: (0, i)),
                  pl.BlockSpec((tn, feat), lambda i: (i, 0))] +
                 [pl.BlockSpec(w.shape, lambda i: (0, 0)) for w in wp],
        out_specs=pl.BlockSpec((num_graphs, 1), lambda i: (0, 0)),
        scratch_shapes=[pltpu.VMEM((num_graphs, feat), F32),
                        pltpu.VMEM((num_graphs, 1), F32)],
        compiler_params=_cparams(("arbitrary",)),
        name="graph_head",
    )(seg, h, *wp)


def _prep_edge_weights(e1w, e1b, e2w, e2b, c1w, c1b, c2w, c2b, *, d, ff):
    h1_raw = e1w.shape[1]
    h1 = _round_up(h1_raw, 128)
    e1wp = jnp.pad(e1w, ((0, 0), (0, h1 - h1_raw)))
    b1p = jnp.pad(e1b, (0, h1 - h1_raw)).reshape(1, h1)
    w1i = e1wp[:d].astype(BF)
    w1j = e1wp[d:2 * d].astype(BF)
    frows = e1wp[2 * d:2 * d + 2 * ff + 1]
    wf = jnp.concatenate(
        [frows, b1p, jnp.zeros((16 - (2 * ff + 2), h1), F32)],
        axis=0).astype(BF)
    w2p = jnp.pad(e2w, ((0, h1 - h1_raw), (0, 0))).astype(BF)
    return [w1i, w1j, wf, w2p, e2b.reshape(1, -1).astype(F32),
            c1w.astype(BF), c1b.reshape(1, -1).astype(F32),
            c2w.reshape(1, -1).astype(F32), c2b.reshape(1, 1).astype(F32)]


def _prep_node_weights(n1w, n1b, n2w, n2b, *, d):
    return [n1w[:d].astype(BF), n1w[d:].astype(BF),
            n1b.reshape(1, -1).astype(F32), n2w.astype(BF),
            n2b.reshape(1, -1).astype(F32)]


def kernel(atomids, identity, coords, edge_index, batch,
           embedding, embedding_id, initialfnn_w, initialfnn_b,
           k0_edge1_w, k0_edge1_b, k0_edge2_w, k0_edge2_b,
           k0_coors1_w, k0_coors1_b, k0_coors2_w, k0_coors2_b,
           k0_node1_w, k0_node1_b, k0_node2_w, k0_node2_b,
           k1_edge1_w, k1_edge1_b, k1_edge2_w, k1_edge2_b,
           k1_coors1_w, k1_coors1_b, k1_coors2_w, k1_coors2_b,
           k1_node1_w, k1_node1_b, k1_node2_w, k1_node2_b,
           k2_edge1_w, k2_edge1_b, k2_edge2_w, k2_edge2_b,
           k2_coors1_w, k2_coors1_b, k2_coors2_w, k2_coors2_b,
           k2_node1_w, k2_node1_b, k2_node2_w, k2_node2_b,
           f0_w, f0_b, f1_w, f1_b, f2_w, f2_b,
           g0_w, g0_b, g1_w, g1_b, g2_w, g2_b):
    d = initialfnn_w.shape[1]
    m_dim = k0_coors1_w.shape[0]
    ff = (k0_edge1_w.shape[0] - 2 * d - 1) // 2
    num_graphs = 64

    src = edge_index[0]
    dst = edge_index[1]
    coors = coords.astype(F32)

    feats, fb16 = _initial_feats(atomids, identity, embedding, embedding_id,
                                 initialfnn_w, initialfnn_b)

    layers = [
        (_prep_edge_weights(k0_edge1_w, k0_edge1_b, k0_edge2_w, k0_edge2_b,
                            k0_coors1_w, k0_coors1_b, k0_coors2_w, k0_coors2_b,
                            d=d, ff=ff),
         _prep_node_weights(k0_node1_w, k0_node1_b, k0_node2_w, k0_node2_b,
                            d=d)),
        (_prep_edge_weights(k1_edge1_w, k1_edge1_b, k1_edge2_w, k1_edge2_b,
                            k1_coors1_w, k1_coors1_b, k1_coors2_w, k1_coors2_b,
                            d=d, ff=ff),
         _prep_node_weights(k1_node1_w, k1_node1_b, k1_node2_w, k1_node2_b,
                            d=d)),
        (_prep_edge_weights(k2_edge1_w, k2_edge1_b, k2_edge2_w, k2_edge2_b,
                            k2_coors1_w, k2_coors1_b, k2_coors2_w, k2_coors2_b,
                            d=d, ff=ff),
         _prep_node_weights(k2_node1_w, k2_node1_b, k2_node2_w, k2_node2_b,
                            d=d)),
    ]

    flist = [feats]
    for ew, nw in layers:
        agg = _edge_aggregate(fb16, coors, src, dst, ew,
                              fourier_features=ff, m_dim=m_dim)
        feats, fb16, coors = _node_update(feats, coors, agg, nw, m_dim=m_dim)
        flist.append(feats)

    fnn_w = [f0_w.astype(BF), f0_b.reshape(1, -1).astype(F32),
             f1_w.astype(BF), f1_b.reshape(1, -1).astype(F32),
             f2_w.astype(BF), f2_b.reshape(1, -1).astype(F32)]
    h = _fnn_stack(flist, fnn_w)

    head_w = [g0_w.astype(BF), g0_b.reshape(1, -1).astype(F32),
              g1_w.astype(BF), g1_b.reshape(1, -1).astype(F32),
              g2_w.reshape(1, -1).astype(F32), g2_b.reshape(1, 1).astype(F32)]
    return _graph_head(h, batch, head_w, num_graphs=num_graphs)


# speedup vs baseline: 2.6441x; 1.8201x over previous
import functools

import jax
import jax.numpy as jnp
from jax import lax
from jax.experimental import pallas as pl
from jax.experimental.pallas import tpu as pltpu

BF = jnp.bfloat16
F32 = jnp.float32
ROW = 384


def _round_up(x, m):
    return ((x + m - 1) // m) * m


def _silu(x):
    return x * jax.nn.sigmoid(x)


def _cparams(sems, vmem=None):
    kw = {"dimension_semantics": sems}
    if vmem is not None:
        kw["vmem_limit_bytes"] = vmem
    return pltpu.CompilerParams(**kw)


def _init_kernel(aid_ref, iid_ref, co_ref, embA_ref, embI_ref, wtop_ref,
                 wbot_ref, b_ref, src_ref, *, n_atom_pad, n_id_pad):
    aid = aid_ref[...]
    iid = iid_ref[...]
    t = aid.shape[0]
    oh_a = (lax.broadcasted_iota(jnp.int32, (t, n_atom_pad), 1) == aid)
    oh_i = (lax.broadcasted_iota(jnp.int32, (t, n_id_pad), 1) == iid)
    p_top = jnp.dot(embI_ref[...], wtop_ref[...], preferred_element_type=F32)
    p_bot = jnp.dot(embA_ref[...], wbot_ref[...], preferred_element_type=F32)
    pre = jnp.dot(oh_i.astype(BF), p_top.astype(BF), preferred_element_type=F32)
    pre = pre + jnp.dot(oh_a.astype(BF), p_bot.astype(BF),
                        preferred_element_type=F32)
    out = _silu(pre + b_ref[...])
    d = out.shape[1]
    src_ref[...] = jnp.concatenate(
        [out, co_ref[...], jnp.zeros((t, ROW - d - 3), F32)], axis=1)


def _initial_src(atomids, identity, coords, embedding, embedding_id, w, b):
    n = atomids.shape[0]
    eid = embedding_id.shape[1]
    na, ni = embedding.shape[0], embedding_id.shape[0]
    na_pad, ni_pad = _round_up(na, 8), _round_up(ni, 8)
    tn = 512 if n % 512 == 0 else n
    embA = jnp.pad(embedding, ((0, na_pad - na), (0, 0))).astype(BF)
    embI = jnp.pad(embedding_id, ((0, ni_pad - ni), (0, 0))).astype(BF)
    wtop = w[:eid].astype(BF)
    wbot = w[eid:].astype(BF)
    bb = b.reshape(1, -1).astype(F32)

    return pl.pallas_call(
        functools.partial(_init_kernel, n_atom_pad=na_pad, n_id_pad=ni_pad),
        out_shape=jax.ShapeDtypeStruct((n, ROW), F32),
        grid=(n // tn,),
        in_specs=[pl.BlockSpec((tn, 1), lambda i: (i, 0)),
                  pl.BlockSpec((tn, 1), lambda i: (i, 0)),
                  pl.BlockSpec((tn, 3), lambda i: (i, 0)),
                  pl.BlockSpec(embA.shape, lambda i: (0, 0)),
                  pl.BlockSpec(embI.shape, lambda i: (0, 0)),
                  pl.BlockSpec(wtop.shape, lambda i: (0, 0)),
                  pl.BlockSpec(wbot.shape, lambda i: (0, 0)),
                  pl.BlockSpec(bb.shape, lambda i: (0, 0))],
        out_specs=pl.BlockSpec((tn, ROW), lambda i: (i, 0)),
        compiler_params=_cparams(("arbitrary",)),
        name="init_feats",
    )(atomids.reshape(n, 1).astype(jnp.int32),
      identity.reshape(n, 1).astype(jnp.int32),
      coords.astype(F32), embA, embI, wtop, wbot, bb)


def _edge_kernel(idx_ref, seg_ref, src_ref,
                 w1i_ref, w1j_ref, wf_ref, w2_ref, b2_ref,
                 wc1_ref, bc1_ref, wc2_ref, bc2_ref,
                 agg_ref, xi_buf, xj_buf,
                 *, fourier_features, n_nodes, m_dim, d, te, n_edges):
    step = pl.program_id(0)

    @pl.when(step == 0)
    def _init():
        agg_ref[...] = jnp.zeros_like(agg_ref)

    base = step * te
    for mi in range(te):
        di = idx_ref[base + mi]
        si = idx_ref[n_edges + base + mi]
        xi_buf[mi, :] = src_ref[di, :]
        xj_buf[mi, :] = src_ref[si, :]

    rows_i = xi_buf[...]
    rows_j = xj_buf[...]
    xi = rows_i[:, :d].astype(BF)
    xj = rows_j[:, :d].astype(BF)
    rel = rows_j[:, d:d + 3] - rows_i[:, d:d + 3]
    d2 = jnp.sum(rel * rel, axis=-1, keepdims=True)

    dks = [d2 * (0.5 ** k) for k in range(fourier_features)]
    ff = jnp.concatenate(
        [jnp.sin(jnp.concatenate(dks, axis=1)),
         jnp.cos(jnp.concatenate(dks, axis=1)), d2, jnp.ones_like(d2),
         jnp.zeros((te, 16 - 2 * fourier_features - 2), F32)],
        axis=1).astype(BF)

    pre = jnp.dot(xi, w1i_ref[...], preferred_element_type=F32)
    pre = pre + jnp.dot(xj, w1j_ref[...], preferred_element_type=F32)
    pre = pre + jnp.dot(ff, wf_ref[...], preferred_element_type=F32)
    h = _silu(pre).astype(BF)

    m_ij = _silu(jnp.dot(h, w2_ref[...], preferred_element_type=F32)
                 + b2_ref[...])
    mb = m_ij.astype(BF)
    ch = _silu(jnp.dot(mb, wc1_ref[...], preferred_element_type=F32)
               + bc1_ref[...])
    cw = jnp.sum(ch * wc2_ref[...], axis=-1, keepdims=True) + bc2_ref[...]

    vals = jnp.concatenate(
        [mb, (cw * rel).astype(BF), jnp.ones((te, 1), BF),
         jnp.zeros((te, 12), BF)], axis=1)

    seg = seg_ref[...]
    one_hot = (lax.broadcasted_iota(jnp.int32, (n_nodes, te), 0)
               == seg).astype(BF)
    agg_ref[0] += jnp.dot(one_hot, vals, preferred_element_type=F32)


def _edge_aggregate(node_src, edge_idx, seg, wp, *, fourier_features,
                    m_dim, d):
    n = node_src.shape[0]
    e = seg.shape[1]
    te = 512
    s = e // te

    agg = pl.pallas_call(
        functools.partial(_edge_kernel, fourier_features=fourier_features,
                          n_nodes=n, m_dim=m_dim, d=d, te=te, n_edges=e),
        out_shape=jax.ShapeDtypeStruct((1, n, m_dim + 16), F32),
        grid=(s,),
        in_specs=[pl.BlockSpec(memory_space=pltpu.SMEM),
                  pl.BlockSpec((1, te), lambda i: (0, i)),
                  pl.BlockSpec((n, ROW), lambda i: (0, 0))] +
                 [pl.BlockSpec(w.shape, lambda i: (0, 0))
                  for w in wp],
        out_specs=pl.BlockSpec((1, n, m_dim + 16), lambda i: (0, 0, 0)),
        scratch_shapes=[pltpu.VMEM((te, ROW), F32),
                        pltpu.VMEM((te, ROW), F32)],
        compiler_params=_cparams(("arbitrary",),
                                 vmem=48 * 1024 * 1024),
        name="edge_agg",
    )(edge_idx, seg, node_src, *wp)
    return agg


def _node_kernel(src_ref, agg_ref,
                 wn1f_ref, wn1m_ref, bn1_ref, wn2_ref, bn2_ref,
                 out_ref, *, m_dim, d):
    agg = agg_ref[0]
    summ = agg[:, :m_dim]
    rest = agg[:, m_dim:]
    cnt = rest[:, 3:4]
    inv = 1.0 / jnp.maximum(cnt, 1.0)
    agg_m = summ * inv
    agg_c = rest[:, :3] * inv

    rows = src_ref[...]
    feats = rows[:, :d]
    t = feats.shape[0]
    pre = jnp.dot(feats.astype(BF), wn1f_ref[...], preferred_element_type=F32)
    pre = pre + jnp.dot(agg_m.astype(BF), wn1m_ref[...],
                        preferred_element_type=F32)
    h = _silu(pre + bn1_ref[...]).astype(BF)
    fo = feats + jnp.dot(h, wn2_ref[...], preferred_element_type=F32) \
        + bn2_ref[...]
    co = rows[:, d:d + 3] + agg_c
    out_ref[...] = jnp.concatenate(
        [fo, co, jnp.zeros((t, ROW - d - 3), F32)], axis=1)


def _node_update(node_src, agg, wp, *, m_dim, d):
    n = node_src.shape[0]
    aw = agg.shape[2]
    tn = 512 if n % 512 == 0 else n

    return pl.pallas_call(
        functools.partial(_node_kernel, m_dim=m_dim, d=d),
        out_shape=jax.ShapeDtypeStruct((n, ROW), F32),
        grid=(n // tn,),
        in_specs=[pl.BlockSpec((tn, ROW), lambda i: (i, 0)),
                  pl.BlockSpec((1, tn, aw), lambda i: (0, i, 0))] +
                 [pl.BlockSpec(w.shape, lambda i: (0, 0)) for w in wp],
        out_specs=pl.BlockSpec((tn, ROW), lambda i: (i, 0)),
        compiler_params=_cparams(("arbitrary",)),
        name="node_update",
    )(node_src, agg, *wp)


def _fnn_kernel(f0_ref, f1_ref, f2_ref, f3_ref,
                w0_ref, b0_ref, w1_ref, b1_ref, w2_ref, b2_ref, o_ref, *, d):
    x = jnp.concatenate(
        [_silu(f0_ref[...][:, :d]), _silu(f1_ref[...][:, :d]),
         _silu(f2_ref[...][:, :d]), _silu(f3_ref[...][:, :d])],
        axis=1).astype(BF)
    h = _silu(jnp.dot(x, w0_ref[...], preferred_element_type=F32)
              + b0_ref[...]).astype(BF)
    h = _silu(jnp.dot(h, w1_ref[...], preferred_element_type=F32)
              + b1_ref[...]).astype(BF)
    h = _silu(jnp.dot(h, w2_ref[...], preferred_element_type=F32)
              + b2_ref[...])
    o_ref[...] = h.astype(BF)


def _fnn_stack(srcs, wp, *, d):
    n = srcs[0].shape[0]
    out_dim = wp[4].shape[1]
    tn = 512 if n % 512 == 0 else n
    return pl.pallas_call(
        functools.partial(_fnn_kernel, d=d),
        out_shape=jax.ShapeDtypeStruct((n, out_dim), BF),
        grid=(n // tn,),
        in_specs=[pl.BlockSpec((tn, ROW), lambda i: (i, 0))
                  for _ in srcs] +
                 [pl.BlockSpec(w.shape, lambda i: (0, 0)) for w in wp],
        out_specs=pl.BlockSpec((tn, out_dim), lambda i: (i, 0)),
        compiler_params=_cparams(("arbitrary",)),
        name="fnn_stack",
    )(*srcs, *wp)


def _head_kernel(seg_ref, h_ref, w0_ref, b0_ref, w1_ref, b1_ref,
                 w2_ref, b2_ref, o_ref, acc_ref, cnt_ref, *, num_graphs):
    step = pl.program_id(0)

    @pl.when(step == 0)
    def _init():
        acc_ref[...] = jnp.zeros_like(acc_ref)
        cnt_ref[...] = jnp.zeros_like(cnt_ref)

    seg = seg_ref[...]
    one_hot = (lax.broadcasted_iota(jnp.int32, (num_graphs, seg.shape[1]), 0)
               == seg)
    acc_ref[...] += jnp.dot(one_hot.astype(BF), h_ref[...],
                            preferred_element_type=F32)
    cnt_ref[...] += jnp.sum(one_hot.astype(F32), axis=-1, keepdims=True)

    @pl.when(step == pl.num_programs(0) - 1)
    def _fin():
        g = acc_ref[...] * (1.0 / jnp.maximum(cnt_ref[...], 1.0))
        g = _silu(jnp.dot(g.astype(BF), w0_ref[...],
                          preferred_element_type=F32) + b0_ref[...])
        g = _silu(jnp.dot(g.astype(BF), w1_ref[...],
                          preferred_element_type=F32) + b1_ref[...])
        o_ref[...] = jnp.sum(g * w2_ref[...], axis=-1, keepdims=True) \
            + b2_ref[...]


def _graph_head(h, batch, wp, *, num_graphs):
    n, feat = h.shape
    tn = 1024 if n % 1024 == 0 else n
    seg = batch.reshape(1, n).astype(jnp.int32)
    return pl.pallas_call(
        functools.partial(_head_kernel, num_graphs=num_graphs),
        out_shape=jax.ShapeDtypeStruct((num_graphs, 1), F32),
        grid=(n // tn,),
        in_specs=[pl.BlockSpec((1, tn), lambda i: (0, i)),
                  pl.BlockSpec((tn, feat), lambda i: (i, 0))] +
                 [pl.BlockSpec(w.shape, lambda i: (0, 0)) for w in wp],
        out_specs=pl.BlockSpec((num_graphs, 1), lambda i: (0, 0)),
        scratch_shapes=[pltpu.VMEM((num_graphs, feat), F32),
                        pltpu.VMEM((num_graphs, 1), F32)],
        compiler_params=_cparams(("arbitrary",)),
        name="graph_head",
    )(seg, h, *wp)


def _prep_edge_weights(e1w, e1b, e2w, e2b, c1w, c1b, c2w, c2b, *, d, ff):
    h1_raw = e1w.shape[1]
    h1 = _round_up(h1_raw, 128)
    e1wp = jnp.pad(e1w, ((0, 0), (0, h1 - h1_raw)))
    b1p = jnp.pad(e1b, (0, h1 - h1_raw)).reshape(1, h1)
    w1i = e1wp[:d].astype(BF)
    w1j = e1wp[d:2 * d].astype(BF)
    frows = e1wp[2 * d:2 * d + 2 * ff + 1]
    wf = jnp.concatenate(
        [frows, b1p, jnp.zeros((16 - (2 * ff + 2), h1), F32)],
        axis=0).astype(BF)
    w2p = jnp.pad(e2w, ((0, h1 - h1_raw), (0, 0))).astype(BF)
    return [w1i, w1j, wf, w2p, e2b.reshape(1, -1).astype(F32),
            c1w.astype(BF), c1b.reshape(1, -1).astype(F32),
            c2w.reshape(1, -1).astype(F32), c2b.reshape(1, 1).astype(F32)]


def _prep_node_weights(n1w, n1b, n2w, n2b, *, d):
    return [n1w[:d].astype(BF), n1w[d:].astype(BF),
            n1b.reshape(1, -1).astype(F32), n2w.astype(BF),
            n2b.reshape(1, -1).astype(F32)]


def kernel(atomids, identity, coords, edge_index, batch,
           embedding, embedding_id, initialfnn_w, initialfnn_b,
           k0_edge1_w, k0_edge1_b, k0_edge2_w, k0_edge2_b,
           k0_coors1_w, k0_coors1_b, k0_coors2_w, k0_coors2_b,
           k0_node1_w, k0_node1_b, k0_node2_w, k0_node2_b,
           k1_edge1_w, k1_edge1_b, k1_edge2_w, k1_edge2_b,
           k1_coors1_w, k1_coors1_b, k1_coors2_w, k1_coors2_b,
           k1_node1_w, k1_node1_b, k1_node2_w, k1_node2_b,
           k2_edge1_w, k2_edge1_b, k2_edge2_w, k2_edge2_b,
           k2_coors1_w, k2_coors1_b, k2_coors2_w, k2_coors2_b,
           k2_node1_w, k2_node1_b, k2_node2_w, k2_node2_b,
           f0_w, f0_b, f1_w, f1_b, f2_w, f2_b,
           g0_w, g0_b, g1_w, g1_b, g2_w, g2_b):
    d = initialfnn_w.shape[1]
    m_dim = k0_coors1_w.shape[0]
    ff = (k0_edge1_w.shape[0] - 2 * d - 1) // 2
    num_graphs = 64

    e = edge_index.shape[1]
    dst = edge_index[1]
    edge_idx = jnp.concatenate([dst, edge_index[0]]).astype(jnp.int32)
    seg = dst.reshape(1, e).astype(jnp.int32)

    node_src = _initial_src(atomids, identity, coords, embedding,
                            embedding_id, initialfnn_w, initialfnn_b)

    layers = [
        (_prep_edge_weights(k0_edge1_w, k0_edge1_b, k0_edge2_w, k0_edge2_b,
                            k0_coors1_w, k0_coors1_b, k0_coors2_w, k0_coors2_b,
                            d=d, ff=ff),
         _prep_node_weights(k0_node1_w, k0_node1_b, k0_node2_w, k0_node2_b,
                            d=d)),
        (_prep_edge_weights(k1_edge1_w, k1_edge1_b, k1_edge2_w, k1_edge2_b,
                            k1_coors1_w, k1_coors1_b, k1_coors2_w, k1_coors2_b,
                            d=d, ff=ff),
         _prep_node_weights(k1_node1_w, k1_node1_b, k1_node2_w, k1_node2_b,
                            d=d)),
        (_prep_edge_weights(k2_edge1_w, k2_edge1_b, k2_edge2_w, k2_edge2_b,
                            k2_coors1_w, k2_coors1_b, k2_coors2_w, k2_coors2_b,
                            d=d, ff=ff),
         _prep_node_weights(k2_node1_w, k2_node1_b, k2_node2_w, k2_node2_b,
                            d=d)),
    ]

    srcs = [node_src]
    for ew, nw in layers:
        agg = _edge_aggregate(node_src, edge_idx, seg, ew,
                              fourier_features=ff, m_dim=m_dim, d=d)
        node_src = _node_update(node_src, agg, nw, m_dim=m_dim, d=d)
        srcs.append(node_src)

    fnn_w = [f0_w.astype(BF), f0_b.reshape(1, -1).astype(F32),
             f1_w.astype(BF), f1_b.reshape(1, -1).astype(F32),
             f2_w.astype(BF), f2_b.reshape(1, -1).astype(F32)]
    h = _fnn_stack(srcs, fnn_w, d=d)

    head_w = [g0_w.astype(BF), g0_b.reshape(1, -1).astype(F32),
              g1_w.astype(BF), g1_b.reshape(1, -1).astype(F32),
              g2_w.reshape(1, -1).astype(F32), g2_b.reshape(1, 1).astype(F32)]
    return _graph_head(h, batch, head_w, num_graphs=num_graphs)


# speedup vs baseline: 3.1004x; 1.1726x over previous
import functools

import jax
import jax.numpy as jnp
from jax import lax
from jax.experimental import pallas as pl
from jax.experimental.pallas import tpu as pltpu

BF = jnp.bfloat16
F32 = jnp.float32
ROW = 384


def _round_up(x, m):
    return ((x + m - 1) // m) * m


def _silu(x):
    return x * jax.nn.sigmoid(x)


def _cparams(sems, vmem=None):
    kw = {"dimension_semantics": sems}
    if vmem is not None:
        kw["vmem_limit_bytes"] = vmem
    return pltpu.CompilerParams(**kw)


def _init_kernel(aid_ref, iid_ref, co_ref, embA_ref, embI_ref, wtop_ref,
                 wbot_ref, b_ref, src_ref, *, n_atom_pad, n_id_pad):
    aid = aid_ref[...]
    iid = iid_ref[...]
    t = aid.shape[0]
    oh_a = (lax.broadcasted_iota(jnp.int32, (t, n_atom_pad), 1) == aid)
    oh_i = (lax.broadcasted_iota(jnp.int32, (t, n_id_pad), 1) == iid)
    p_top = jnp.dot(embI_ref[...], wtop_ref[...], preferred_element_type=F32)
    p_bot = jnp.dot(embA_ref[...], wbot_ref[...], preferred_element_type=F32)
    pre = jnp.dot(oh_i.astype(BF), p_top.astype(BF), preferred_element_type=F32)
    pre = pre + jnp.dot(oh_a.astype(BF), p_bot.astype(BF),
                        preferred_element_type=F32)
    out = _silu(pre + b_ref[...])
    d = out.shape[1]
    src_ref[...] = jnp.concatenate(
        [out, co_ref[...], jnp.zeros((t, ROW - d - 3), F32)], axis=1)


def _initial_src(atomids, identity, coords, embedding, embedding_id, w, b):
    n = atomids.shape[0]
    eid = embedding_id.shape[1]
    na, ni = embedding.shape[0], embedding_id.shape[0]
    na_pad, ni_pad = _round_up(na, 8), _round_up(ni, 8)
    tn = 512 if n % 512 == 0 else n
    embA = jnp.pad(embedding, ((0, na_pad - na), (0, 0))).astype(BF)
    embI = jnp.pad(embedding_id, ((0, ni_pad - ni), (0, 0))).astype(BF)
    wtop = w[:eid].astype(BF)
    wbot = w[eid:].astype(BF)
    bb = b.reshape(1, -1).astype(F32)

    return pl.pallas_call(
        functools.partial(_init_kernel, n_atom_pad=na_pad, n_id_pad=ni_pad),
        out_shape=jax.ShapeDtypeStruct((n, ROW), F32),
        grid=(n // tn,),
        in_specs=[pl.BlockSpec((tn, 1), lambda i: (i, 0)),
                  pl.BlockSpec((tn, 1), lambda i: (i, 0)),
                  pl.BlockSpec((tn, 3), lambda i: (i, 0)),
                  pl.BlockSpec(embA.shape, lambda i: (0, 0)),
                  pl.BlockSpec(embI.shape, lambda i: (0, 0)),
                  pl.BlockSpec(wtop.shape, lambda i: (0, 0)),
                  pl.BlockSpec(wbot.shape, lambda i: (0, 0)),
                  pl.BlockSpec(bb.shape, lambda i: (0, 0))],
        out_specs=pl.BlockSpec((tn, ROW), lambda i: (i, 0)),
        compiler_params=_cparams(("arbitrary",)),
        name="init_feats",
    )(atomids.reshape(n, 1).astype(jnp.int32),
      identity.reshape(n, 1).astype(jnp.int32),
      coords.astype(F32), embA, embI, wtop, wbot, bb)


def _edge_kernel(idx_ref, seg_ref, src_ref,
                 w1i_ref, w1j_ref, wf_ref, w2_ref, b2_ref,
                 wc1_ref, bc1_ref, wc2_ref, bc2_ref,
                 agg_ref, xi_buf, xj_buf,
                 *, fourier_features, n_nodes, m_dim, d, te, n_edges):
    step = pl.program_id(0)

    @pl.when(step == 0)
    def _init():
        agg_ref[...] = jnp.zeros_like(agg_ref)

    base = step * te
    for mi in range(te):
        di = idx_ref[base + mi]
        si = idx_ref[n_edges + base + mi]
        xi_buf[mi, :] = src_ref[di, :]
        xj_buf[mi, :] = src_ref[si, :]

    rows_i = xi_buf[...]
    rows_j = xj_buf[...]
    xi = rows_i[:, :d].astype(BF)
    xj = rows_j[:, :d].astype(BF)
    rel = rows_j[:, d:d + 3] - rows_i[:, d:d + 3]
    d2 = jnp.sum(rel * rel, axis=-1, keepdims=True)

    d2_row = lax.transpose(d2, (1, 0))
    dk_rows = [d2_row * (0.5 ** k) for k in range(fourier_features)]
    dk_dense = jnp.concatenate(dk_rows, axis=0)
    fft = jnp.concatenate(
        [jnp.sin(dk_dense), jnp.cos(dk_dense), d2_row,
         jnp.ones_like(d2_row),
         jnp.zeros((16 - 2 * fourier_features - 2, te), F32)],
        axis=0).astype(BF)

    pre = jnp.dot(xi, w1i_ref[...], preferred_element_type=F32)
    pre = pre + jnp.dot(xj, w1j_ref[...], preferred_element_type=F32)
    pre = pre + lax.dot_general(fft, wf_ref[...],
                                (((0,), (0,)), ((), ())),
                                preferred_element_type=F32)
    h = _silu(pre).astype(BF)

    m_ij = _silu(jnp.dot(h, w2_ref[...], preferred_element_type=F32)
                 + b2_ref[...])
    mb = m_ij.astype(BF)
    ch = _silu(jnp.dot(mb, wc1_ref[...], preferred_element_type=F32)
               + bc1_ref[...])
    cw = jnp.sum(ch * wc2_ref[...], axis=-1, keepdims=True) + bc2_ref[...]

    vals = jnp.concatenate(
        [mb, (cw * rel).astype(BF), jnp.ones((te, 1), BF),
         jnp.zeros((te, 12), BF)], axis=1)

    seg = seg_ref[...]
    one_hot = (lax.broadcasted_iota(jnp.int32, (n_nodes, te), 0)
               == seg).astype(BF)
    agg_ref[0] += jnp.dot(one_hot, vals, preferred_element_type=F32)


def _edge_aggregate(node_src, edge_idx, seg, wp, *, fourier_features,
                    m_dim, d):
    n = node_src.shape[0]
    e = seg.shape[1]
    te = 1024 if e % 1024 == 0 else 512
    s = e // te

    agg = pl.pallas_call(
        functools.partial(_edge_kernel, fourier_features=fourier_features,
                          n_nodes=n, m_dim=m_dim, d=d, te=te, n_edges=e),
        out_shape=jax.ShapeDtypeStruct((1, n, m_dim + 16), F32),
        grid=(s,),
        in_specs=[pl.BlockSpec(memory_space=pltpu.SMEM),
                  pl.BlockSpec((1, te), lambda i: (0, i)),
                  pl.BlockSpec((n, ROW), lambda i: (0, 0))] +
                 [pl.BlockSpec(w.shape, lambda i: (0, 0))
                  for w in wp],
        out_specs=pl.BlockSpec((1, n, m_dim + 16), lambda i: (0, 0, 0)),
        scratch_shapes=[pltpu.VMEM((te, ROW), F32),
                        pltpu.VMEM((te, ROW), F32)],
        compiler_params=_cparams(("arbitrary",),
                                 vmem=52 * 1024 * 1024),
        name="edge_agg",
    )(edge_idx, seg, node_src, *wp)
    return agg


def _node_kernel(src_ref, agg_ref,
                 wn1f_ref, wn1m_ref, bn1_ref, wn2_ref, bn2_ref,
                 out_ref, *, m_dim, d):
    agg = agg_ref[0]
    summ = agg[:, :m_dim]
    rest = agg[:, m_dim:]
    cnt = rest[:, 3:4]
    inv = 1.0 / jnp.maximum(cnt, 1.0)
    agg_m = summ * inv
    agg_c = rest[:, :3] * inv

    rows = src_ref[...]
    feats = rows[:, :d]
    t = feats.shape[0]
    pre = jnp.dot(feats.astype(BF), wn1f_ref[...], preferred_element_type=F32)
    pre = pre + jnp.dot(agg_m.astype(BF), wn1m_ref[...],
                        preferred_element_type=F32)
    h = _silu(pre + bn1_ref[...]).astype(BF)
    fo = feats + jnp.dot(h, wn2_ref[...], preferred_element_type=F32) \
        + bn2_ref[...]
    co = rows[:, d:d + 3] + agg_c
    out_ref[...] = jnp.concatenate(
        [fo, co, jnp.zeros((t, ROW - d - 3), F32)], axis=1)


def _node_update(node_src, agg, wp, *, m_dim, d):
    n = node_src.shape[0]
    aw = agg.shape[2]
    tn = 512 if n % 512 == 0 else n

    return pl.pallas_call(
        functools.partial(_node_kernel, m_dim=m_dim, d=d),
        out_shape=jax.ShapeDtypeStruct((n, ROW), F32),
        grid=(n // tn,),
        in_specs=[pl.BlockSpec((tn, ROW), lambda i: (i, 0)),
                  pl.BlockSpec((1, tn, aw), lambda i: (0, i, 0))] +
                 [pl.BlockSpec(w.shape, lambda i: (0, 0)) for w in wp],
        out_specs=pl.BlockSpec((tn, ROW), lambda i: (i, 0)),
        compiler_params=_cparams(("arbitrary",)),
        name="node_update",
    )(node_src, agg, *wp)


def _fnn_kernel(f0_ref, f1_ref, f2_ref, f3_ref,
                w0_ref, b0_ref, w1_ref, b1_ref, w2_ref, b2_ref, o_ref, *, d):
    x = jnp.concatenate(
        [_silu(f0_ref[...][:, :d]), _silu(f1_ref[...][:, :d]),
         _silu(f2_ref[...][:, :d]), _silu(f3_ref[...][:, :d])],
        axis=1).astype(BF)
    h = _silu(jnp.dot(x, w0_ref[...], preferred_element_type=F32)
              + b0_ref[...]).astype(BF)
    h = _silu(jnp.dot(h, w1_ref[...], preferred_element_type=F32)
              + b1_ref[...]).astype(BF)
    h = _silu(jnp.dot(h, w2_ref[...], preferred_element_type=F32)
              + b2_ref[...])
    o_ref[...] = h.astype(BF)


def _fnn_stack(srcs, wp, *, d):
    n = srcs[0].shape[0]
    out_dim = wp[4].shape[1]
    tn = 512 if n % 512 == 0 else n
    return pl.pallas_call(
        functools.partial(_fnn_kernel, d=d),
        out_shape=jax.ShapeDtypeStruct((n, out_dim), BF),
        grid=(n // tn,),
        in_specs=[pl.BlockSpec((tn, ROW), lambda i: (i, 0))
                  for _ in srcs] +
                 [pl.BlockSpec(w.shape, lambda i: (0, 0)) for w in wp],
        out_specs=pl.BlockSpec((tn, out_dim), lambda i: (i, 0)),
        compiler_params=_cparams(("arbitrary",)),
        name="fnn_stack",
    )(*srcs, *wp)


def _head_kernel(seg_ref, h_ref, w0_ref, b0_ref, w1_ref, b1_ref,
                 w2_ref, b2_ref, o_ref, acc_ref, cnt_ref, *, num_graphs):
    step = pl.program_id(0)

    @pl.when(step == 0)
    def _init():
        acc_ref[...] = jnp.zeros_like(acc_ref)
        cnt_ref[...] = jnp.zeros_like(cnt_ref)

    seg = seg_ref[...]
    one_hot = (lax.broadcasted_iota(jnp.int32, (num_graphs, seg.shape[1]), 0)
               == seg)
    acc_ref[...] += jnp.dot(one_hot.astype(BF), h_ref[...],
                            preferred_element_type=F32)
    cnt_ref[...] += jnp.sum(one_hot.astype(F32), axis=-1, keepdims=True)

    @pl.when(step == pl.num_programs(0) - 1)
    def _fin():
        g = acc_ref[...] * (1.0 / jnp.maximum(cnt_ref[...], 1.0))
        g = _silu(jnp.dot(g.astype(BF), w0_ref[...],
                          preferred_element_type=F32) + b0_ref[...])
        g = _silu(jnp.dot(g.astype(BF), w1_ref[...],
                          preferred_element_type=F32) + b1_ref[...])
        o_ref[...] = jnp.sum(g * w2_ref[...], axis=-1, keepdims=True) \
            + b2_ref[...]


def _graph_head(h, batch, wp, *, num_graphs):
    n, feat = h.shape
    tn = 1024 if n % 1024 == 0 else n
    seg = batch.reshape(1, n).astype(jnp.int32)
    return pl.pallas_call(
        functools.partial(_head_kernel, num_graphs=num_graphs),
        out_shape=jax.ShapeDtypeStruct((num_graphs, 1), F32),
        grid=(n // tn,),
        in_specs=[pl.BlockSpec((1, tn), lambda i: (0, i)),
                  pl.BlockSpec((tn, feat), lambda i: (i, 0))] +
                 [pl.BlockSpec(w.shape, lambda i: (0, 0)) for w in wp],
        out_specs=pl.BlockSpec((num_graphs, 1), lambda i: (0, 0)),
        scratch_shapes=[pltpu.VMEM((num_graphs, feat), F32),
                        pltpu.VMEM((num_graphs, 1), F32)],
        compiler_params=_cparams(("arbitrary",)),
        name="graph_head",
    )(seg, h, *wp)


def _prep_edge_weights(e1w, e1b, e2w, e2b, c1w, c1b, c2w, c2b, *, d, ff):
    h1_raw = e1w.shape[1]
    h1 = _round_up(h1_raw, 128)
    e1wp = jnp.pad(e1w, ((0, 0), (0, h1 - h1_raw)))
    b1p = jnp.pad(e1b, (0, h1 - h1_raw)).reshape(1, h1)
    w1i = e1wp[:d].astype(BF)
    w1j = e1wp[d:2 * d].astype(BF)
    frows = e1wp[2 * d:2 * d + 2 * ff + 1]
    wf = jnp.concatenate(
        [frows, b1p, jnp.zeros((16 - (2 * ff + 2), h1), F32)],
        axis=0).astype(BF)
    w2p = jnp.pad(e2w, ((0, h1 - h1_raw), (0, 0))).astype(BF)
    return [w1i, w1j, wf, w2p, e2b.reshape(1, -1).astype(F32),
            c1w.astype(BF), c1b.reshape(1, -1).astype(F32),
            c2w.reshape(1, -1).astype(F32), c2b.reshape(1, 1).astype(F32)]


def _prep_node_weights(n1w, n1b, n2w, n2b, *, d):
    return [n1w[:d].astype(BF), n1w[d:].astype(BF),
            n1b.reshape(1, -1).astype(F32), n2w.astype(BF),
            n2b.reshape(1, -1).astype(F32)]


def kernel(atomids, identity, coords, edge_index, batch,
           embedding, embedding_id, initialfnn_w, initialfnn_b,
           k0_edge1_w, k0_edge1_b, k0_edge2_w, k0_edge2_b,
           k0_coors1_w, k0_coors1_b, k0_coors2_w, k0_coors2_b,
           k0_node1_w, k0_node1_b, k0_node2_w, k0_node2_b,
           k1_edge1_w, k1_edge1_b, k1_edge2_w, k1_edge2_b,
           k1_coors1_w, k1_coors1_b, k1_coors2_w, k1_coors2_b,
           k1_node1_w, k1_node1_b, k1_node2_w, k1_node2_b,
           k2_edge1_w, k2_edge1_b, k2_edge2_w, k2_edge2_b,
           k2_coors1_w, k2_coors1_b, k2_coors2_w, k2_coors2_b,
           k2_node1_w, k2_node1_b, k2_node2_w, k2_node2_b,
           f0_w, f0_b, f1_w, f1_b, f2_w, f2_b,
           g0_w, g0_b, g1_w, g1_b, g2_w, g2_b):
    d = initialfnn_w.shape[1]
    m_dim = k0_coors1_w.shape[0]
    ff = (k0_edge1_w.shape[0] - 2 * d - 1) // 2
    num_graphs = 64

    e = edge_index.shape[1]
    dst = edge_index[1]
    edge_idx = jnp.concatenate([dst, edge_index[0]]).astype(jnp.int32)
    seg = dst.reshape(1, e).astype(jnp.int32)

    node_src = _initial_src(atomids, identity, coords, embedding,
                            embedding_id, initialfnn_w, initialfnn_b)

    layers = [
        (_prep_edge_weights(k0_edge1_w, k0_edge1_b, k0_edge2_w, k0_edge2_b,
                            k0_coors1_w, k0_coors1_b, k0_coors2_w, k0_coors2_b,
                            d=d, ff=ff),
         _prep_node_weights(k0_node1_w, k0_node1_b, k0_node2_w, k0_node2_b,
                            d=d)),
        (_prep_edge_weights(k1_edge1_w, k1_edge1_b, k1_edge2_w, k1_edge2_b,
                            k1_coors1_w, k1_coors1_b, k1_coors2_w, k1_coors2_b,
                            d=d, ff=ff),
         _prep_node_weights(k1_node1_w, k1_node1_b, k1_node2_w, k1_node2_b,
                            d=d)),
        (_prep_edge_weights(k2_edge1_w, k2_edge1_b, k2_edge2_w, k2_edge2_b,
                            k2_coors1_w, k2_coors1_b, k2_coors2_w, k2_coors2_b,
                            d=d, ff=ff),
         _prep_node_weights(k2_node1_w, k2_node1_b, k2_node2_w, k2_node2_b,
                            d=d)),
    ]

    srcs = [node_src]
    for ew, nw in layers:
        agg = _edge_aggregate(node_src, edge_idx, seg, ew,
                              fourier_features=ff, m_dim=m_dim, d=d)
        node_src = _node_update(node_src, agg, nw, m_dim=m_dim, d=d)
        srcs.append(node_src)

    fnn_w = [f0_w.astype(BF), f0_b.reshape(1, -1).astype(F32),
             f1_w.astype(BF), f1_b.reshape(1, -1).astype(F32),
             f2_w.astype(BF), f2_b.reshape(1, -1).astype(F32)]
    h = _fnn_stack(srcs, fnn_w, d=d)

    head_w = [g0_w.astype(BF), g0_b.reshape(1, -1).astype(F32),
              g1_w.astype(BF), g1_b.reshape(1, -1).astype(F32),
              g2_w.reshape(1, -1).astype(F32), g2_b.reshape(1, 1).astype(F32)]
    return _graph_head(h, batch, head_w, num_graphs=num_graphs)


# speedup vs baseline: 3.2357x; 1.0436x over previous
import functools

import jax
import jax.numpy as jnp
from jax import lax
from jax.experimental import pallas as pl
from jax.experimental.pallas import tpu as pltpu

BF = jnp.bfloat16
F32 = jnp.float32
ROW = 384


def _round_up(x, m):
    return ((x + m - 1) // m) * m


def _silu(x):
    return x * jax.nn.sigmoid(x)


def _cparams(sems, vmem=None):
    kw = {"dimension_semantics": sems}
    if vmem is not None:
        kw["vmem_limit_bytes"] = vmem
    return pltpu.CompilerParams(**kw)


def _init_kernel(aid_ref, iid_ref, co_ref, embA_ref, embI_ref, wtop_ref,
                 wbot_ref, b_ref, src_ref, *, n_atom_pad, n_id_pad):
    aid = aid_ref[...]
    iid = iid_ref[...]
    t = aid.shape[0]
    oh_a = (lax.broadcasted_iota(jnp.int32, (t, n_atom_pad), 1) == aid)
    oh_i = (lax.broadcasted_iota(jnp.int32, (t, n_id_pad), 1) == iid)
    p_top = jnp.dot(embI_ref[...], wtop_ref[...], preferred_element_type=F32)
    p_bot = jnp.dot(embA_ref[...], wbot_ref[...], preferred_element_type=F32)
    pre = jnp.dot(oh_i.astype(BF), p_top.astype(BF), preferred_element_type=F32)
    pre = pre + jnp.dot(oh_a.astype(BF), p_bot.astype(BF),
                        preferred_element_type=F32)
    out = _silu(pre + b_ref[...])
    d = out.shape[1]
    src_ref[...] = jnp.concatenate(
        [out, co_ref[...], jnp.zeros((t, ROW - d - 3), F32)], axis=1)


def _initial_src(atomids, identity, coords, embedding, embedding_id, w, b):
    n = atomids.shape[0]
    eid = embedding_id.shape[1]
    na, ni = embedding.shape[0], embedding_id.shape[0]
    na_pad, ni_pad = _round_up(na, 8), _round_up(ni, 8)
    tn = 512 if n % 512 == 0 else n
    embA = jnp.pad(embedding, ((0, na_pad - na), (0, 0))).astype(BF)
    embI = jnp.pad(embedding_id, ((0, ni_pad - ni), (0, 0))).astype(BF)
    wtop = w[:eid].astype(BF)
    wbot = w[eid:].astype(BF)
    bb = b.reshape(1, -1).astype(F32)

    return pl.pallas_call(
        functools.partial(_init_kernel, n_atom_pad=na_pad, n_id_pad=ni_pad),
        out_shape=jax.ShapeDtypeStruct((n, ROW), F32),
        grid=(n // tn,),
        in_specs=[pl.BlockSpec((tn, 1), lambda i: (i, 0)),
                  pl.BlockSpec((tn, 1), lambda i: (i, 0)),
                  pl.BlockSpec((tn, 3), lambda i: (i, 0)),
                  pl.BlockSpec(embA.shape, lambda i: (0, 0)),
                  pl.BlockSpec(embI.shape, lambda i: (0, 0)),
                  pl.BlockSpec(wtop.shape, lambda i: (0, 0)),
                  pl.BlockSpec(wbot.shape, lambda i: (0, 0)),
                  pl.BlockSpec(bb.shape, lambda i: (0, 0))],
        out_specs=pl.BlockSpec((tn, ROW), lambda i: (i, 0)),
        compiler_params=_cparams(("arbitrary",)),
        name="init_feats",
    )(atomids.reshape(n, 1).astype(jnp.int32),
      identity.reshape(n, 1).astype(jnp.int32),
      coords.astype(F32), embA, embI, wtop, wbot, bb)


def _layers_kernel(idx_ref, seg_ref, src0_ref,
                   w1i_ref, w1j_ref, wf_ref, w2_ref, b2_ref,
                   wc1_ref, bc1_ref, wc2_ref, bc2_ref,
                   wn1f_ref, wn1m_ref, bn1_ref, wn2_ref, bn2_ref,
                   hist_ref, src_cur, agg_ref, xi_buf, xj_buf,
                   *, fourier_features, n_nodes, m_dim, d, te, n_edges,
                   n_steps):
    lyr = pl.program_id(0)
    stp = pl.program_id(1)

    @pl.when((lyr == 0) & (stp == 0))
    def _load_src():
        src_cur[...] = src0_ref[...]

    @pl.when(stp == 0)
    def _zero_agg():
        agg_ref[...] = jnp.zeros_like(agg_ref)

    base = stp * te
    for mi in range(te):
        di = idx_ref[base + mi]
        si = idx_ref[n_edges + base + mi]
        xi_buf[mi, :] = src_cur[di, :]
        xj_buf[mi, :] = src_cur[si, :]

    rows_i = xi_buf[...]
    rows_j = xj_buf[...]
    xi = rows_i[:, :d].astype(BF)
    xj = rows_j[:, :d].astype(BF)
    rel = rows_j[:, d:d + 3] - rows_i[:, d:d + 3]
    d2 = jnp.sum(rel * rel, axis=-1, keepdims=True)

    d2_row = lax.transpose(d2, (1, 0))
    dk_rows = [d2_row * (0.5 ** k) for k in range(fourier_features)]
    dk_dense = jnp.concatenate(dk_rows, axis=0)
    fft = jnp.concatenate(
        [jnp.sin(dk_dense), jnp.cos(dk_dense), d2_row,
         jnp.ones_like(d2_row),
         jnp.zeros((16 - 2 * fourier_features - 2, te), F32)],
        axis=0).astype(BF)

    pre = jnp.dot(xi, w1i_ref[0], preferred_element_type=F32)
    pre = pre + jnp.dot(xj, w1j_ref[0], preferred_element_type=F32)
    pre = pre + lax.dot_general(fft, wf_ref[0],
                                (((0,), (0,)), ((), ())),
                                preferred_element_type=F32)
    h = _silu(pre).astype(BF)

    m_ij = _silu(jnp.dot(h, w2_ref[0], preferred_element_type=F32)
                 + b2_ref[0])
    mb = m_ij.astype(BF)
    ch = _silu(jnp.dot(mb, wc1_ref[0], preferred_element_type=F32)
               + bc1_ref[0])
    cw = jnp.sum(ch * wc2_ref[0], axis=-1, keepdims=True) + bc2_ref[0]

    vals = jnp.concatenate(
        [mb, (cw * rel).astype(BF), jnp.ones((te, 1), BF),
         jnp.zeros((te, 12), BF)], axis=1)

    seg = seg_ref[...]
    one_hot = (lax.broadcasted_iota(jnp.int32, (n_nodes, te), 0)
               == seg).astype(BF)
    agg_ref[...] += jnp.dot(one_hot, vals, preferred_element_type=F32)

    @pl.when(stp == n_steps - 1)
    def _node_update():
        agg = agg_ref[...]
        summ = agg[:, :m_dim]
        rest = agg[:, m_dim:]
        cnt = rest[:, 3:4]
        inv = 1.0 / jnp.maximum(cnt, 1.0)
        agg_m = summ * inv
        agg_c = rest[:, :3] * inv

        rows = src_cur[...]
        feats = rows[:, :d]
        npre = jnp.dot(feats.astype(BF), wn1f_ref[0],
                       preferred_element_type=F32)
        npre = npre + jnp.dot(agg_m.astype(BF), wn1m_ref[0],
                              preferred_element_type=F32)
        nh = _silu(npre + bn1_ref[0]).astype(BF)
        fo = feats + jnp.dot(nh, wn2_ref[0], preferred_element_type=F32) \
            + bn2_ref[0]
        co = rows[:, d:d + 3] + agg_c
        new_src = jnp.concatenate(
            [fo, co, jnp.zeros((n_nodes, ROW - d - 3), F32)], axis=1)
        src_cur[...] = new_src
        hist_ref[0] = new_src


def _egnn_layers(node_src, edge_idx, seg, ews, nws, *, fourier_features,
                 m_dim, d):
    n = node_src.shape[0]
    e = seg.shape[1]
    te = 1024 if e % 1024 == 0 else 512
    s = e // te
    n_layers = len(ews)

    stacked = [jnp.stack(ws) for ws in zip(*[ew + nw
                                             for ew, nw in zip(ews, nws)])]
    wspecs = [pl.BlockSpec((1,) + w.shape[1:], lambda l, i: (l, 0, 0))
              for w in stacked]

    hist = pl.pallas_call(
        functools.partial(_layers_kernel, fourier_features=fourier_features,
                          n_nodes=n, m_dim=m_dim, d=d, te=te, n_edges=e,
                          n_steps=s),
        out_shape=jax.ShapeDtypeStruct((n_layers, n, ROW), F32),
        grid=(n_layers, s),
        in_specs=[pl.BlockSpec(memory_space=pltpu.SMEM),
                  pl.BlockSpec((1, te), lambda l, i: (0, i)),
                  pl.BlockSpec((n, ROW), lambda l, i: (0, 0))] + wspecs,
        out_specs=pl.BlockSpec((1, n, ROW), lambda l, i: (l, 0, 0)),
        scratch_shapes=[pltpu.VMEM((n, ROW), F32),
                        pltpu.VMEM((n, m_dim + 16), F32),
                        pltpu.VMEM((te, ROW), F32),
                        pltpu.VMEM((te, ROW), F32)],
        compiler_params=_cparams(("arbitrary", "arbitrary"),
                                 vmem=52 * 1024 * 1024),
        name="egnn_layers",
    )(edge_idx, seg, node_src, *stacked)
    return hist


def _fnn_kernel(f0_ref, f1_ref, f2_ref, f3_ref,
                w0_ref, b0_ref, w1_ref, b1_ref, w2_ref, b2_ref, o_ref, *, d):
    x = jnp.concatenate(
        [_silu(f0_ref[...][:, :d]), _silu(f1_ref[...][:, :d]),
         _silu(f2_ref[...][:, :d]), _silu(f3_ref[...][:, :d])],
        axis=1).astype(BF)
    h = _silu(jnp.dot(x, w0_ref[...], preferred_element_type=F32)
              + b0_ref[...]).astype(BF)
    h = _silu(jnp.dot(h, w1_ref[...], preferred_element_type=F32)
              + b1_ref[...]).astype(BF)
    h = _silu(jnp.dot(h, w2_ref[...], preferred_element_type=F32)
              + b2_ref[...])
    o_ref[...] = h.astype(BF)


def _fnn_stack(srcs, wp, *, d):
    n = srcs[0].shape[0]
    out_dim = wp[4].shape[1]
    tn = 512 if n % 512 == 0 else n
    return pl.pallas_call(
        functools.partial(_fnn_kernel, d=d),
        out_shape=jax.ShapeDtypeStruct((n, out_dim), BF),
        grid=(n // tn,),
        in_specs=[pl.BlockSpec((tn, ROW), lambda i: (i, 0))
                  for _ in srcs] +
                 [pl.BlockSpec(w.shape, lambda i: (0, 0)) for w in wp],
        out_specs=pl.BlockSpec((tn, out_dim), lambda i: (i, 0)),
        compiler_params=_cparams(("arbitrary",)),
        name="fnn_stack",
    )(*srcs, *wp)


def _head_kernel(seg_ref, h_ref, w0_ref, b0_ref, w1_ref, b1_ref,
                 w2_ref, b2_ref, o_ref, acc_ref, cnt_ref, *, num_graphs):
    step = pl.program_id(0)

    @pl.when(step == 0)
    def _init():
        acc_ref[...] = jnp.zeros_like(acc_ref)
        cnt_ref[...] = jnp.zeros_like(cnt_ref)

    seg = seg_ref[...]
    one_hot = (lax.broadcasted_iota(jnp.int32, (num_graphs, seg.shape[1]), 0)
               == seg)
    acc_ref[...] += jnp.dot(one_hot.astype(BF), h_ref[...],
                            preferred_element_type=F32)
    cnt_ref[...] += jnp.sum(one_hot.astype(F32), axis=-1, keepdims=True)

    @pl.when(step == pl.num_programs(0) - 1)
    def _fin():
        g = acc_ref[...] * (1.0 / jnp.maximum(cnt_ref[...], 1.0))
        g = _silu(jnp.dot(g.astype(BF), w0_ref[...],
                          preferred_element_type=F32) + b0_ref[...])
        g = _silu(jnp.dot(g.astype(BF), w1_ref[...],
                          preferred_element_type=F32) + b1_ref[...])
        o_ref[...] = jnp.sum(g * w2_ref[...], axis=-1, keepdims=True) \
            + b2_ref[...]


def _graph_head(h, batch, wp, *, num_graphs):
    n, feat = h.shape
    tn = 1024 if n % 1024 == 0 else n
    seg = batch.reshape(1, n).astype(jnp.int32)
    return pl.pallas_call(
        functools.partial(_head_kernel, num_graphs=num_graphs),
        out_shape=jax.ShapeDtypeStruct((num_graphs, 1), F32),
        grid=(n // tn,),
        in_specs=[pl.BlockSpec((1, tn), lambda i: (0, i)),
                  pl.BlockSpec((tn, feat), lambda i: (i, 0))] +
                 [pl.BlockSpec(w.shape, lambda i: (0, 0)) for w in wp],
        out_specs=pl.BlockSpec((num_graphs, 1), lambda i: (0, 0)),
        scratch_shapes=[pltpu.VMEM((num_graphs, feat), F32),
                        pltpu.VMEM((num_graphs, 1), F32)],
        compiler_params=_cparams(("arbitrary",)),
        name="graph_head",
    )(seg, h, *wp)


def _prep_edge_weights(e1w, e1b, e2w, e2b, c1w, c1b, c2w, c2b, *, d, ff):
    h1_raw = e1w.shape[1]
    h1 = _round_up(h1_raw, 128)
    e1wp = jnp.pad(e1w, ((0, 0), (0, h1 - h1_raw)))
    b1p = jnp.pad(e1b, (0, h1 - h1_raw)).reshape(1, h1)
    w1i = e1wp[:d].astype(BF)
    w1j = e1wp[d:2 * d].astype(BF)
    frows = e1wp[2 * d:2 * d + 2 * ff + 1]
    wf = jnp.concatenate(
        [frows, b1p, jnp.zeros((16 - (2 * ff + 2), h1), F32)],
        axis=0).astype(BF)
    w2p = jnp.pad(e2w, ((0, h1 - h1_raw), (0, 0))).astype(BF)
    return [w1i, w1j, wf, w2p, e2b.reshape(1, -1).astype(F32),
            c1w.astype(BF), c1b.reshape(1, -1).astype(F32),
            c2w.reshape(1, -1).astype(F32), c2b.reshape(1, 1).astype(F32)]


def _prep_node_weights(n1w, n1b, n2w, n2b, *, d):
    return [n1w[:d].astype(BF), n1w[d:].astype(BF),
            n1b.reshape(1, -1).astype(F32), n2w.astype(BF),
            n2b.reshape(1, -1).astype(F32)]


def kernel(atomids, identity, coords, edge_index, batch,
           embedding, embedding_id, initialfnn_w, initialfnn_b,
           k0_edge1_w, k0_edge1_b, k0_edge2_w, k0_edge2_b,
           k0_coors1_w, k0_coors1_b, k0_coors2_w, k0_coors2_b,
           k0_node1_w, k0_node1_b, k0_node2_w, k0_node2_b,
           k1_edge1_w, k1_edge1_b, k1_edge2_w, k1_edge2_b,
           k1_coors1_w, k1_coors1_b, k1_coors2_w, k1_coors2_b,
           k1_node1_w, k1_node1_b, k1_node2_w, k1_node2_b,
           k2_edge1_w, k2_edge1_b, k2_edge2_w, k2_edge2_b,
           k2_coors1_w, k2_coors1_b, k2_coors2_w, k2_coors2_b,
           k2_node1_w, k2_node1_b, k2_node2_w, k2_node2_b,
           f0_w, f0_b, f1_w, f1_b, f2_w, f2_b,
           g0_w, g0_b, g1_w, g1_b, g2_w, g2_b):
    d = initialfnn_w.shape[1]
    m_dim = k0_coors1_w.shape[0]
    ff = (k0_edge1_w.shape[0] - 2 * d - 1) // 2
    num_graphs = 64

    e = edge_index.shape[1]
    dst = edge_index[1]
    edge_idx = jnp.concatenate([dst, edge_index[0]]).astype(jnp.int32)
    seg = dst.reshape(1, e).astype(jnp.int32)

    node_src = _initial_src(atomids, identity, coords, embedding,
                            embedding_id, initialfnn_w, initialfnn_b)

    layers = [
        (_prep_edge_weights(k0_edge1_w, k0_edge1_b, k0_edge2_w, k0_edge2_b,
                            k0_coors1_w, k0_coors1_b, k0_coors2_w, k0_coors2_b,
                            d=d, ff=ff),
         _prep_node_weights(k0_node1_w, k0_node1_b, k0_node2_w, k0_node2_b,
                            d=d)),
        (_prep_edge_weights(k1_edge1_w, k1_edge1_b, k1_edge2_w, k1_edge2_b,
                            k1_coors1_w, k1_coors1_b, k1_coors2_w, k1_coors2_b,
                            d=d, ff=ff),
         _prep_node_weights(k1_node1_w, k1_node1_b, k1_node2_w, k1_node2_b,
                            d=d)),
        (_prep_edge_weights(k2_edge1_w, k2_edge1_b, k2_edge2_w, k2_edge2_b,
                            k2_coors1_w, k2_coors1_b, k2_coors2_w, k2_coors2_b,
                            d=d, ff=ff),
         _prep_node_weights(k2_node1_w, k2_node1_b, k2_node2_w, k2_node2_b,
                            d=d)),
    ]

    hist = _egnn_layers(node_src, edge_idx, seg,
                        [ew for ew, _ in layers], [nw for _, nw in layers],
                        fourier_features=ff, m_dim=m_dim, d=d)
    srcs = [node_src] + [hist[i] for i in range(len(layers))]

    fnn_w = [f0_w.astype(BF), f0_b.reshape(1, -1).astype(F32),
             f1_w.astype(BF), f1_b.reshape(1, -1).astype(F32),
             f2_w.astype(BF), f2_b.reshape(1, -1).astype(F32)]
    h = _fnn_stack(srcs, fnn_w, d=d)

    head_w = [g0_w.astype(BF), g0_b.reshape(1, -1).astype(F32),
              g1_w.astype(BF), g1_b.reshape(1, -1).astype(F32),
              g2_w.reshape(1, -1).astype(F32), g2_b.reshape(1, 1).astype(F32)]
    return _graph_head(h, batch, head_w, num_graphs=num_graphs)
